# baseline (speedup 1.0000x reference)
import functools

import jax
import jax.numpy as jnp
from jax import lax
from jax.experimental import pallas as pl
from jax.experimental.pallas import tpu as pltpu
from jax.experimental.pallas import tpu_sc as plsc

BATCH = 4096
LENGTH = 200
DIM = 32
B = BATCH * LENGTH
NVOC = 1000000
NC, NS = 2, 16
NW = NC * NS
BPW = B // NW
CHUNK = 1600
NCHUNK = BPW // CHUNK

TC = 800
TOT_TCH = NVOC // TC

_mesh = plsc.VectorSubcoreMesh(
    core_axis_name="c", subcore_axis_name="s", num_cores=NC, num_subcores=NS
)


@functools.partial(
    pl.kernel,
    out_type=jax.ShapeDtypeStruct((NVOC, DIM), jnp.float32),
    mesh=_mesh,
    scratch_types=[
        pltpu.VMEM((DIM, TC), jnp.float32),
        pltpu.VMEM((DIM, TC), jnp.float32),
        pltpu.VMEM((TC, DIM + 1), jnp.float32),
        pltpu.VMEM((TC, DIM + 1), jnp.float32),
        pltpu.SemaphoreType.DMA,
        pltpu.SemaphoreType.DMA,
        pltpu.SemaphoreType.DMA,
        pltpu.SemaphoreType.DMA,
    ],
    compiler_params=pltpu.CompilerParams(
        use_tc_tiling_on_sc=False, needs_layout_passes=False
    ),
)
def _transpose_kernel(tt_hbm, out_hbm, in_a, in_b, tr_a, tr_b, si_a, si_b,
                      so_a, so_b):
    ins = (in_a, in_b)
    trs = (tr_a, tr_b)
    si = (si_a, si_b)
    so = (so_a, so_b)

    wid = lax.axis_index("s") * NC + lax.axis_index("c")
    nj = (TOT_TCH - 1 - wid) // NW + 1
    lanes = lax.iota(jnp.int32, 16)

    def in_slice(j):
        c0 = (j * NW + wid) * TC
        return tt_hbm.at[:, pl.ds(c0, TC)]

    def out_slice(j):
        c0 = (j * NW + wid) * TC
        return out_hbm.at[pl.ds(c0, TC), :]

    def transpose_buf(b):
        def body(c16, carry):
            c0 = c16 * 16
            cvec = lanes + c0
            for f in range(DIM):
                v = ins[b][f, pl.ds(c0, 16)]
                plsc.store_scatter(trs[b], [cvec, jnp.full((16,), f, jnp.int32)], v)
            return carry
        lax.fori_loop(0, TC // 16, body, 0)

    def fetch(j, b):
        pltpu.async_copy(in_slice(j), ins[b], si[b])

    def flush(j, b):
        pltpu.async_copy(trs[b].at[:, : DIM], out_slice(j), so[b])

    def step(j, carry):
        b = lax.rem(j, 2)

        def phase(bb):
            pltpu.make_async_copy(in_slice(j), ins[bb], si[bb]).wait()

            @pl.when(j + 1 < nj)
            def _():
                pltpu.async_copy(in_slice(j + 1), ins[1 - bb], si[1 - bb])

            @pl.when(j >= 2)
            def _():
                pltpu.make_async_copy(
                    trs[bb].at[:, : DIM], out_slice(j - 2), so[bb]
                ).wait()

            transpose_buf(bb)
            flush(j, bb)

        @pl.when(b == 0)
        def _():
            phase(0)

        @pl.when(b == 1)
        def _():
            phase(1)

        return carry

    fetch(0, 0)
    lax.fori_loop(0, nj, step, 0)

    for par in (0, 1):
        @pl.when(lax.rem(nj, 2) == par)
        def _(par=par):
            pltpu.make_async_copy(
                trs[par].at[:, : DIM], out_slice(nj - 2), so[par]
            ).wait()
            pltpu.make_async_copy(
                trs[1 - par].at[:, : DIM], out_slice(nj - 1), so[1 - par]
            ).wait()


@functools.partial(
    pl.kernel,
    out_type=jax.ShapeDtypeStruct((B, DIM), jnp.float32),
    mesh=_mesh,
    scratch_types=[
        pltpu.VMEM((CHUNK,), jnp.int32),
        pltpu.VMEM((CHUNK,), jnp.int32),
        pltpu.VMEM((CHUNK, DIM), jnp.float32),
        pltpu.VMEM((CHUNK, DIM), jnp.float32),
        pltpu.SemaphoreType.DMA,
        pltpu.SemaphoreType.DMA,
        pltpu.SemaphoreType.DMA,
        pltpu.SemaphoreType.DMA,
        pltpu.SemaphoreType.DMA,
        pltpu.SemaphoreType.DMA,
    ],
    compiler_params=pltpu.CompilerParams(use_tc_tiling_on_sc=False),
)
def _gather_kernel(ids_hbm, table_hbm, out_hbm, idx_a, idx_b, rows_a,
                   rows_b, si_a, si_b, sg_a, sg_b, so_a, so_b):
    idx = (idx_a, idx_b)
    rows = (rows_a, rows_b)
    si = (si_a, si_b)
    sg = (sg_a, sg_b)
    so = (so_a, so_b)

    wid = lax.axis_index("s") * NC + lax.axis_index("c")
    base = wid * BPW

    def ids_slice(j):
        return ids_hbm.at[pl.ds(base + j * CHUNK, CHUNK)]

    def out_slice(j):
        return out_hbm.at[pl.ds(base + j * CHUNK, CHUNK)]

    pltpu.async_copy(ids_slice(0), idx[0], si[0])

    for j in range(NCHUNK):
        b = j % 2
        o = (j + 1) % 2
        pltpu.make_async_copy(ids_slice(j), idx[b], si[b]).wait()
        if j >= 2:
            pltpu.make_async_copy(rows[b], out_slice(j - 2), so[b]).wait()
        pltpu.async_copy(table_hbm.at[idx[b]], rows[b], sg[b])
        if j >= 1:
            pltpu.make_async_copy(table_hbm.at[idx[o]], rows[o], sg[o]).wait()
            pltpu.async_copy(rows[o], out_slice(j - 1), so[o])
        if j + 1 < NCHUNK:
            pltpu.async_copy(ids_slice(j + 1), idx[o], si[o])

    last = NCHUNK - 1
    lb = last % 2
    lo = (last + 1) % 2
    pltpu.make_async_copy(table_hbm.at[idx[lb]], rows[lb], sg[lb]).wait()
    pltpu.async_copy(rows[lb], out_slice(last), so[lb])
    pltpu.make_async_copy(rows[lo], out_slice(last - 1), so[lo]).wait()
    pltpu.make_async_copy(rows[lb], out_slice(last), so[lb]).wait()


def kernel(inputs, embeddings):
    ids = jnp.reshape(inputs, (B,)).astype(jnp.int32)
    table_rm = _transpose_kernel(embeddings.T)
    out = _gather_kernel(ids, table_rm)
    return jnp.reshape(out, (BATCH, LENGTH, DIM))


# speedup vs baseline: 3.4475x; 3.4475x over previous
import functools

import jax
import jax.numpy as jnp
from jax import lax
from jax.experimental import pallas as pl
from jax.experimental.pallas import tpu as pltpu
from jax.experimental.pallas import tpu_sc as plsc

BATCH = 4096
LENGTH = 200
DIM = 32
B = BATCH * LENGTH
NC, NS = 2, 16
NW = NC * NS
BPW = B // NW
CHUNK = 1600
NCHUNK = BPW // CHUNK

_mesh = plsc.VectorSubcoreMesh(
    core_axis_name="c", subcore_axis_name="s", num_cores=NC, num_subcores=NS
)


@functools.partial(
    pl.kernel,
    out_type=jax.ShapeDtypeStruct((B, DIM), jnp.float32),
    mesh=_mesh,
    scratch_types=[
        pltpu.VMEM((CHUNK,), jnp.int32),
        pltpu.VMEM((CHUNK,), jnp.int32),
        pltpu.VMEM((CHUNK, DIM), jnp.float32),
        pltpu.VMEM((CHUNK, DIM), jnp.float32),
        pltpu.SemaphoreType.DMA,
        pltpu.SemaphoreType.DMA,
        pltpu.SemaphoreType.DMA,
        pltpu.SemaphoreType.DMA,
        pltpu.SemaphoreType.DMA,
        pltpu.SemaphoreType.DMA,
    ],
    compiler_params=pltpu.CompilerParams(use_tc_tiling_on_sc=False),
)
def _gather_kernel(ids_hbm, table_hbm, out_hbm, idx_a, idx_b, rows_a,
                   rows_b, si_a, si_b, sg_a, sg_b, so_a, so_b):
    idx = (idx_a, idx_b)
    rows = (rows_a, rows_b)
    si = (si_a, si_b)
    sg = (sg_a, sg_b)
    so = (so_a, so_b)

    wid = lax.axis_index("s") * NC + lax.axis_index("c")
    base = wid * BPW

    def ids_slice(j):
        return ids_hbm.at[pl.ds(base + j * CHUNK, CHUNK)]

    def out_slice(j):
        return out_hbm.at[pl.ds(base + j * CHUNK, CHUNK)]

    pltpu.async_copy(ids_slice(0), idx[0], si[0])

    for j in range(NCHUNK):
        b = j % 2
        o = (j + 1) % 2
        pltpu.make_async_copy(ids_slice(j), idx[b], si[b]).wait()
        if j >= 2:
            pltpu.make_async_copy(rows[b], out_slice(j - 2), so[b]).wait()
        pltpu.async_copy(table_hbm.at[idx[b]], rows[b], sg[b])
        if j >= 1:
            pltpu.make_async_copy(table_hbm.at[idx[o]], rows[o], sg[o]).wait()
            pltpu.async_copy(rows[o], out_slice(j - 1), so[o])
        if j + 1 < NCHUNK:
            pltpu.async_copy(ids_slice(j + 1), idx[o], si[o])

    last = NCHUNK - 1
    lb = last % 2
    lo = (last + 1) % 2
    pltpu.make_async_copy(table_hbm.at[idx[lb]], rows[lb], sg[lb]).wait()
    pltpu.async_copy(rows[lb], out_slice(last), so[lb])
    pltpu.make_async_copy(rows[lo], out_slice(last - 1), so[lo]).wait()
    pltpu.make_async_copy(rows[lb], out_slice(last), so[lb]).wait()


def kernel(inputs, embeddings):
    ids = jnp.reshape(inputs, (B,)).astype(jnp.int32)
    out = _gather_kernel(ids, embeddings)
    return jnp.reshape(out, (BATCH, LENGTH, DIM))


# speedup vs baseline: 3.4492x; 1.0005x over previous
import functools

import jax
import jax.numpy as jnp
from jax import lax
from jax.experimental import pallas as pl
from jax.experimental.pallas import tpu as pltpu
from jax.experimental.pallas import tpu_sc as plsc

BATCH = 4096
LENGTH = 200
DIM = 32
B = BATCH * LENGTH
HALF = B // 2
NC, NS = 2, 16
NW = NC * NS
BPW = HALF // NW
CHUNK = 1600
NCHUNK = BPW // CHUNK

_mesh = plsc.VectorSubcoreMesh(
    core_axis_name="c", subcore_axis_name="s", num_cores=NC, num_subcores=NS
)


@functools.partial(
    pl.kernel,
    out_type=jax.ShapeDtypeStruct((HALF, DIM), jnp.float32),
    mesh=_mesh,
    scratch_types=[
        pltpu.VMEM((CHUNK,), jnp.int32),
        pltpu.VMEM((CHUNK,), jnp.int32),
        pltpu.VMEM((CHUNK, DIM), jnp.float32),
        pltpu.VMEM((CHUNK, DIM), jnp.float32),
        pltpu.SemaphoreType.DMA,
        pltpu.SemaphoreType.DMA,
        pltpu.SemaphoreType.DMA,
        pltpu.SemaphoreType.DMA,
        pltpu.SemaphoreType.DMA,
        pltpu.SemaphoreType.DMA,
    ],
    compiler_params=pltpu.CompilerParams(use_tc_tiling_on_sc=False),
)
def _gather_kernel(ids_hbm, table_hbm, out_hbm, idx_a, idx_b, rows_a,
                   rows_b, si_a, si_b, sg_a, sg_b, so_a, so_b):
    idx = (idx_a, idx_b)
    rows = (rows_a, rows_b)
    si = (si_a, si_b)
    sg = (sg_a, sg_b)
    so = (so_a, so_b)

    wid = lax.axis_index("s") * NC + lax.axis_index("c")
    base = wid * BPW

    def ids_slice(j):
        return ids_hbm.at[pl.ds(base + j * CHUNK, CHUNK)]

    def out_slice(j):
        return out_hbm.at[pl.ds(base + j * CHUNK, CHUNK)]

    pltpu.async_copy(ids_slice(0), idx[0], si[0])

    for j in range(NCHUNK):
        b = j % 2
        o = (j + 1) % 2
        pltpu.make_async_copy(ids_slice(j), idx[b], si[b]).wait()
        if j >= 2:
            pltpu.make_async_copy(rows[b], out_slice(j - 2), so[b]).wait()
        pltpu.async_copy(table_hbm.at[idx[b]], rows[b], sg[b])
        if j >= 1:
            pltpu.make_async_copy(table_hbm.at[idx[o]], rows[o], sg[o]).wait()
            pltpu.async_copy(rows[o], out_slice(j - 1), so[o])
        if j + 1 < NCHUNK:
            pltpu.async_copy(ids_slice(j + 1), idx[o], si[o])

    last = NCHUNK - 1
    lb = last % 2
    lo = (last + 1) % 2
    pltpu.make_async_copy(table_hbm.at[idx[lb]], rows[lb], sg[lb]).wait()
    pltpu.async_copy(rows[lb], out_slice(last), so[lb])
    pltpu.make_async_copy(rows[lo], out_slice(last - 1), so[lo]).wait()
    pltpu.make_async_copy(rows[lb], out_slice(last), so[lb]).wait()


def kernel(inputs, embeddings):
    ids = jnp.reshape(inputs, (B,)).astype(jnp.int32)
    out1 = _gather_kernel(ids[:HALF], embeddings)
    out2 = _gather_kernel(ids[HALF:], embeddings)
    half_shape = (BATCH // 2, LENGTH, DIM)
    return jnp.concatenate(
        [jnp.reshape(out1, half_shape), jnp.reshape(out2, half_shape)], axis=0
    )
